# baseline (speedup 1.0000x reference)
import jax
import jax.numpy as jnp
from jax.experimental import pallas as pl

_N_LAYERS = 2
_ITERS = 2
_BJ = 8
_LOG2E = 1.4426950408889634
_CLAMP = 100.0


def _mpnn_body(jets_ref, W_emb_ref, b_emb_ref, W_h_ref, W_m_ref, b_mp_ref,
               W_ro_ref, out_ref):
    W_emb = W_emb_ref[...]
    b_emb = b_emb_ref[...]
    W_ro = W_ro_ref[...]
    O = W_ro.shape[1] // 2
    H = W_emb.shape[1]
    s = jnp.sqrt(jnp.float32(_LOG2E) / jnp.sqrt(jnp.float32(H)))

    ones_fh = jnp.ones((W_emb.shape[0], H), jnp.bfloat16)
    g = [None] * _BJ
    mvf = [None] * _BJ
    mvb = [None] * _BJ
    for j in range(_BJ):
        x = jets_ref[j]
        xb = x.astype(jnp.bfloat16)
        asum = jnp.dot(jnp.abs(xb), ones_fh,
                       preferred_element_type=jnp.float32)
        mvf[j] = (asum > 0.0).astype(jnp.float32)
        mvb[j] = mvf[j].astype(jnp.bfloat16)
        e = jnp.dot(x, W_emb, preferred_element_type=jnp.float32)
        g[j] = (jnp.tanh(e + b_emb) * s) * mvf[j]

    for r in range(_N_LAYERS * _ITERS):
        i = r // _ITERS
        W_h = W_h_ref[i]
        W_m = W_m_ref[i]
        b_mp = b_mp_ref[i]
        P = [None] * _BJ
        gb = [None] * _BJ
        for j in range(_BJ):
            gb[j] = g[j].astype(jnp.bfloat16)
            l2 = jax.lax.dot_general(
                gb[j], gb[j], (((1,), (1,)), ((), ())),
                preferred_element_type=jnp.float32)
            P[j] = jnp.exp2(jnp.minimum(l2, _CLAMP)).astype(jnp.bfloat16)
        for j in range(_BJ):
            agg = jnp.dot(P[j], gb[j], preferred_element_type=jnp.float32)
            dn = jnp.dot(P[j], mvb[j], preferred_element_type=jnp.float32)
            m = agg / jnp.maximum(dn, 1e-30)
            g[j] = jax.nn.relu(
                jnp.dot(g[j], W_h, preferred_element_type=jnp.float32)
                + jnp.dot(m, W_m, preferred_element_type=jnp.float32)
                + b_mp) * mvf[j]

    for j in range(_BJ):
        ro = jnp.dot(g[j], W_ro, preferred_element_type=jnp.float32)
        gate = jax.nn.sigmoid(ro[:, :O])
        o = jnp.sum(gate * ro[:, O:], axis=0, keepdims=True)
        out_ref[j:j + 1, :] = o


def kernel(jets, W_emb, b_emb, W_h, W_m, b_mp, W_gate, W_out, b_out):
    B, N, F = jets.shape
    H = W_emb.shape[1]
    O = W_gate.shape[1]
    s = jnp.sqrt(jnp.float32(_LOG2E) / jnp.sqrt(jnp.float32(H)))
    b_emb2 = b_emb.reshape(1, H)
    b_mp3 = (b_mp * s).reshape(b_mp.shape[0], 1, H)
    W_ro = jnp.concatenate([W_gate, W_out], axis=1) / s
    full = lambda shape: pl.BlockSpec(shape, lambda i: (0,) * len(shape))
    out = pl.pallas_call(
        _mpnn_body,
        grid=(B // _BJ,),
        in_specs=[
            pl.BlockSpec((_BJ, N, F), lambda i: (i, 0, 0)),
            full((F, H)),
            full((1, H)),
            full(W_h.shape),
            full(W_m.shape),
            full(b_mp3.shape),
            full(W_ro.shape),
        ],
        out_specs=pl.BlockSpec((_BJ, O), lambda i: (i, 0)),
        out_shape=jax.ShapeDtypeStruct((B, O), jnp.float32),
    )(jets, W_emb, b_emb2,
      W_h, W_m, b_mp3, W_ro)
    return out + b_out


# speedup vs baseline: 1.3632x; 1.3632x over previous
import jax
import jax.numpy as jnp
from jax.experimental import pallas as pl

_N_LAYERS = 2
_ITERS = 2
_BJ = 8
_LOG2E = 1.4426950408889634
_CLAMP = 100.0


def _mpnn_body(jets_ref, W_emb_ref, b_emb_ref, W_h_ref, W_m_ref, b_mp_ref,
               W_ro_ref, out_ref):
    W_emb = W_emb_ref[...]
    b_emb = b_emb_ref[...]
    W_ro = W_ro_ref[...]
    O = W_ro.shape[1] // 2
    H = W_emb.shape[1]
    s = jnp.sqrt(jnp.float32(_LOG2E) / jnp.sqrt(jnp.float32(H)))

    g = [None] * _BJ
    mvf = [None] * _BJ
    mvb = [None] * _BJ
    for j in range(_BJ):
        x = jets_ref[j]
        mv = (jnp.sum(jnp.abs(x), axis=1, keepdims=True) > 0.0).astype(
            jnp.float32)
        mvf[j] = jnp.broadcast_to(mv, (x.shape[0], H))
        mvb[j] = mvf[j].astype(jnp.bfloat16)
        e = jnp.dot(x, W_emb, preferred_element_type=jnp.float32)
        g[j] = (jnp.tanh(e + b_emb) * s) * mvf[j]

    for r in range(_N_LAYERS * _ITERS):
        i = r // _ITERS
        W_h = W_h_ref[i]
        W_mb = W_m_ref[i]
        b_mp = b_mp_ref[i]
        P = [None] * _BJ
        q = [None] * _BJ
        for j in range(_BJ):
            gb = g[j].astype(jnp.bfloat16)
            q[j] = jnp.dot(gb, W_mb,
                           preferred_element_type=jnp.float32).astype(
                               jnp.bfloat16)
            l2 = jax.lax.dot_general(
                gb, gb, (((1,), (1,)), ((), ())),
                preferred_element_type=jnp.float32)
            P[j] = jnp.exp2(jnp.minimum(l2, _CLAMP)).astype(jnp.bfloat16)
        for j in range(_BJ):
            t = jnp.dot(P[j], q[j], preferred_element_type=jnp.float32)
            dn = jnp.dot(P[j], mvb[j], preferred_element_type=jnp.float32)
            g[j] = jax.nn.relu(
                jnp.dot(g[j], W_h, preferred_element_type=jnp.float32)
                + t / jnp.maximum(dn, 1e-30)
                + b_mp) * mvf[j]

    for j in range(_BJ):
        ro = jnp.dot(g[j], W_ro, preferred_element_type=jnp.float32)
        gate = jax.nn.sigmoid(ro[:, :O])
        o = jnp.sum(gate * ro[:, O:], axis=0, keepdims=True)
        out_ref[j:j + 1, :] = o


def kernel(jets, W_emb, b_emb, W_h, W_m, b_mp, W_gate, W_out, b_out):
    B, N, F = jets.shape
    H = W_emb.shape[1]
    O = W_gate.shape[1]
    s = jnp.sqrt(jnp.float32(_LOG2E) / jnp.sqrt(jnp.float32(H)))
    b_emb2 = b_emb.reshape(1, H)
    b_mp3 = (b_mp * s).reshape(b_mp.shape[0], 1, H)
    W_ro = jnp.concatenate([W_gate, W_out], axis=1) / s
    full = lambda shape: pl.BlockSpec(shape, lambda i: (0,) * len(shape))
    out = pl.pallas_call(
        _mpnn_body,
        grid=(B // _BJ,),
        in_specs=[
            pl.BlockSpec((_BJ, N, F), lambda i: (i, 0, 0)),
            full((F, H)),
            full((1, H)),
            full(W_h.shape),
            full(W_m.shape),
            full(b_mp3.shape),
            full(W_ro.shape),
        ],
        out_specs=pl.BlockSpec((_BJ, O), lambda i: (i, 0)),
        out_shape=jax.ShapeDtypeStruct((B, O), jnp.float32),
    )(jets, W_emb, b_emb2, W_h, W_m.astype(jnp.bfloat16), b_mp3, W_ro)
    return out + b_out


# speedup vs baseline: 1.5888x; 1.1656x over previous
import jax
import jax.numpy as jnp
from jax.experimental import pallas as pl

_N_LAYERS = 2
_ITERS = 2
_BJ = 8
_LOG2E = 1.4426950408889634
_CLAMP = 100.0


def _mpnn_body(jets_ref, W_emb_ref, b_emb_ref, W_h_ref, W_m_ref, b_mp_ref,
               W_ro_ref, out_ref):
    W_emb = W_emb_ref[...]
    b_emb = b_emb_ref[...]
    W_ro = W_ro_ref[...]
    O = W_ro.shape[1] // 2
    H = W_emb.shape[1]
    s = jnp.sqrt(jnp.float32(_LOG2E) / jnp.sqrt(jnp.float32(H)))

    g = [None] * _BJ
    mvf = [None] * _BJ
    nm = [None] * _BJ
    for j in range(_BJ):
        x = jets_ref[j]
        mv = (jnp.sum(jnp.abs(x), axis=1, keepdims=True) > 0.0).astype(
            jnp.float32)
        mvf[j] = jnp.broadcast_to(mv, (x.shape[0], H))
        nm[j] = jnp.float32(x.shape[0]) - jnp.sum(mv)
        e = jnp.dot(x, W_emb, preferred_element_type=jnp.float32)
        g[j] = (jnp.tanh(e + b_emb) * s) * mvf[j]

    for r in range(_N_LAYERS * _ITERS):
        i = r // _ITERS
        W_h = W_h_ref[i]
        W_mb = W_m_ref[i]
        b_mp = b_mp_ref[i]
        P = [None] * _BJ
        q = [None] * _BJ
        dn = [None] * _BJ
        for j in range(_BJ):
            gb = g[j].astype(jnp.bfloat16)
            q[j] = jnp.dot(gb, W_mb,
                           preferred_element_type=jnp.float32).astype(
                               jnp.bfloat16)
            l2 = jax.lax.dot_general(
                gb, gb, (((1,), (1,)), ((), ())),
                preferred_element_type=jnp.float32)
            Pf = jnp.exp2(jnp.minimum(l2, _CLAMP))
            dn[j] = jnp.sum(Pf, axis=1, keepdims=True) - nm[j]
            P[j] = Pf.astype(jnp.bfloat16)
        for j in range(_BJ):
            t = jnp.dot(P[j], q[j], preferred_element_type=jnp.float32)
            g[j] = jax.nn.relu(
                jnp.dot(g[j], W_h, preferred_element_type=jnp.float32)
                + t / jnp.maximum(dn[j], 1e-30)
                + b_mp) * mvf[j]

    for j in range(_BJ):
        ro = jnp.dot(g[j], W_ro, preferred_element_type=jnp.float32)
        gate = jax.nn.sigmoid(ro[:, :O])
        o = jnp.sum(gate * ro[:, O:], axis=0, keepdims=True)
        out_ref[j:j + 1, :] = o


def kernel(jets, W_emb, b_emb, W_h, W_m, b_mp, W_gate, W_out, b_out):
    B, N, F = jets.shape
    H = W_emb.shape[1]
    O = W_gate.shape[1]
    s = jnp.sqrt(jnp.float32(_LOG2E) / jnp.sqrt(jnp.float32(H)))
    b_emb2 = b_emb.reshape(1, H)
    b_mp3 = (b_mp * s).reshape(b_mp.shape[0], 1, H)
    W_ro = jnp.concatenate([W_gate, W_out], axis=1) / s
    full = lambda shape: pl.BlockSpec(shape, lambda i: (0,) * len(shape))
    out = pl.pallas_call(
        _mpnn_body,
        grid=(B // _BJ,),
        in_specs=[
            pl.BlockSpec((_BJ, N, F), lambda i: (i, 0, 0)),
            full((F, H)),
            full((1, H)),
            full(W_h.shape),
            full(W_m.shape),
            full(b_mp3.shape),
            full(W_ro.shape),
        ],
        out_specs=pl.BlockSpec((_BJ, O), lambda i: (i, 0)),
        out_shape=jax.ShapeDtypeStruct((B, O), jnp.float32),
    )(jets, W_emb, b_emb2, W_h, W_m.astype(jnp.bfloat16), b_mp3, W_ro)

    return out + b_out


# speedup vs baseline: 1.6083x; 1.0123x over previous
import jax
import jax.numpy as jnp
from jax.experimental import pallas as pl

_N_LAYERS = 2
_ITERS = 2
_BJ = 8
_LOG2E = 1.4426950408889634
_CLAMP = 100.0


def _mpnn_body(jets_ref, W_emb_ref, b_emb_ref, W_h_ref, W_m_ref, b_mp_ref,
               W_ro_ref, out_ref):
    W_emb = W_emb_ref[...]
    b_emb = b_emb_ref[...]
    W_ro = W_ro_ref[...]
    O = W_ro.shape[1] // 2
    H = W_emb.shape[1]
    s = jnp.sqrt(jnp.float32(_LOG2E) / jnp.sqrt(jnp.float32(H)))

    ones_fh = jnp.ones((W_emb.shape[0], H), jnp.bfloat16)
    g = [None] * _BJ
    mvf = [None] * _BJ
    nm = [None] * _BJ
    for j in range(_BJ):
        x = jets_ref[j]
        N = x.shape[1]
        asum = jax.lax.dot_general(
            jnp.abs(x).astype(jnp.bfloat16), ones_fh,
            (((0,), (0,)), ((), ())),
            preferred_element_type=jnp.float32)
        mvf[j] = (asum > 0.0).astype(jnp.float32)
        nm[j] = jnp.float32(N) - jnp.sum(mvf[j]) / jnp.float32(H)
        e = jax.lax.dot_general(
            x, W_emb, (((0,), (0,)), ((), ())),
            preferred_element_type=jnp.float32)
        g[j] = (jnp.tanh(e + b_emb) * s) * mvf[j]

    for r in range(_N_LAYERS * _ITERS):
        i = r // _ITERS
        W_h = W_h_ref[i]
        W_mb = W_m_ref[i]
        b_mp = b_mp_ref[i]
        P = [None] * _BJ
        q = [None] * _BJ
        dn = [None] * _BJ
        for j in range(_BJ):
            gb = g[j].astype(jnp.bfloat16)
            q[j] = jnp.dot(gb, W_mb,
                           preferred_element_type=jnp.float32).astype(
                               jnp.bfloat16)
            l2 = jax.lax.dot_general(
                gb, gb, (((1,), (1,)), ((), ())),
                preferred_element_type=jnp.float32)
            Pf = jnp.exp2(jnp.minimum(l2, _CLAMP))
            dn[j] = jnp.sum(Pf, axis=1, keepdims=True) - nm[j]
            P[j] = Pf.astype(jnp.bfloat16)
        for j in range(_BJ):
            t = jnp.dot(P[j], q[j], preferred_element_type=jnp.float32)
            g[j] = jax.nn.relu(
                jnp.dot(g[j], W_h, preferred_element_type=jnp.float32)
                + t / jnp.maximum(dn[j], 1e-30)
                + b_mp) * mvf[j]

    for j in range(_BJ):
        ro = jnp.dot(g[j], W_ro, preferred_element_type=jnp.float32)
        gate = jax.nn.sigmoid(ro[:, :O])
        o = jnp.sum(gate * ro[:, O:], axis=0, keepdims=True)
        out_ref[j:j + 1, :] = o


def kernel(jets, W_emb, b_emb, W_h, W_m, b_mp, W_gate, W_out, b_out):
    B, N, F = jets.shape
    H = W_emb.shape[1]
    O = W_gate.shape[1]
    s = jnp.sqrt(jnp.float32(_LOG2E) / jnp.sqrt(jnp.float32(H)))
    b_emb2 = b_emb.reshape(1, H)
    b_mp3 = (b_mp * s).reshape(b_mp.shape[0], 1, H)
    W_ro = jnp.concatenate([W_gate, W_out], axis=1) / s
    jets_t = jnp.swapaxes(jets, 1, 2)
    full = lambda shape: pl.BlockSpec(shape, lambda i: (0,) * len(shape))
    out = pl.pallas_call(
        _mpnn_body,
        grid=(B // _BJ,),
        in_specs=[
            pl.BlockSpec((_BJ, F, N), lambda i: (i, 0, 0)),
            full((F, H)),
            full((1, H)),
            full(W_h.shape),
            full(W_m.shape),
            full(b_mp3.shape),
            full(W_ro.shape),
        ],
        out_specs=pl.BlockSpec((_BJ, O), lambda i: (i, 0)),
        out_shape=jax.ShapeDtypeStruct((B, O), jnp.float32),
    )(jets_t, W_emb, b_emb2, W_h, W_m.astype(jnp.bfloat16), b_mp3, W_ro)
    return out + b_out
